# baseline (speedup 1.0000x reference)
import functools

import jax
import jax.numpy as jnp
from jax import lax
from jax.experimental import pallas as pl
from jax.experimental.pallas import tpu as pltpu, tpu_sc as plsc

_BATCH = 16384
_DIM = 64
_ROWS = 100001
_HALF = 50048
_BMAIN = 49920
_TAIL0 = _HALF + _BMAIN
_CHUNK = 4096
_NCHUNK = _BATCH // _CHUNK
_NL = 16


@jax.jit
def _lookup(die_idx, table_t, tail_t):
    info = plsc.get_sparse_core_info()
    nw = info.num_cores * info.num_subcores
    rows_per_w = _DIM // nw
    assert rows_per_w == 2 and info.num_lanes == _NL

    mesh = plsc.VectorSubcoreMesh(core_axis_name="c", subcore_axis_name="s")

    @functools.partial(
        pl.kernel,
        mesh=mesh,
        compiler_params=pltpu.CompilerParams(needs_layout_passes=False),
        out_type=jax.ShapeDtypeStruct((_DIM, _BATCH), jnp.float32),
        scratch_types=[
            pltpu.VMEM((1, _HALF), jnp.float32),
            pltpu.VMEM((1, _HALF), jnp.float32),
            pltpu.VMEM((_BATCH,), jnp.int32),
            pltpu.VMEM((1, _CHUNK), jnp.float32),
            pltpu.VMEM((1, _CHUNK), jnp.float32),
            pltpu.SemaphoreType.DMA,
            pltpu.SemaphoreType.DMA,
            pltpu.SemaphoreType.DMA,
            pltpu.SemaphoreType.DMA,
        ],
    )
    def k(idx_hbm, table_hbm, tail_hbm, out_hbm, row_a, row_b, idx_v, out0,
          out1, sem_a, sem_b, sem_w0, sem_w1):
        wid = lax.axis_index("s") * info.num_cores + lax.axis_index("c")
        zv = jnp.zeros((_NL,), jnp.int32)
        outc = (out0, out1)
        sem_w = (sem_w0, sem_w1)

        def fire_a(r):
            pltpu.make_async_copy(
                table_hbm.at[pl.ds(r, 1), pl.ds(0, _HALF)], row_a, sem_a
            ).start()

        def wait_a():
            pltpu.make_async_copy(
                table_hbm.at[pl.ds(0, 1), pl.ds(0, _HALF)], row_a, sem_a
            ).wait()

        def fire_b(r):
            pltpu.make_async_copy(
                table_hbm.at[pl.ds(r, 1), pl.ds(_HALF, _BMAIN)],
                row_b.at[:, pl.ds(0, _BMAIN)],
                sem_b,
            ).start()
            pltpu.make_async_copy(
                tail_hbm.at[pl.ds(r, 1), :],
                row_b.at[:, pl.ds(_BMAIN, 128)],
                sem_b,
            ).start()

        def wait_b():
            pltpu.make_async_copy(
                table_hbm.at[pl.ds(0, 1), pl.ds(_HALF, _BMAIN)],
                row_b.at[:, pl.ds(0, _BMAIN)],
                sem_b,
            ).wait()
            pltpu.make_async_copy(
                tail_hbm.at[pl.ds(0, 1), :],
                row_b.at[:, pl.ds(_BMAIN, 128)],
                sem_b,
            ).wait()

        def pass_a(chunk):
            buf = outc[chunk % 2]

            def body(g, carry):
                base = chunk * _CHUNK + g * _NL
                iv = idx_v[pl.ds(base, _NL)]
                ivc = jnp.minimum(iv, _HALF - 1)
                buf[0, pl.ds(g * _NL, _NL)] = plsc.load_gather(
                    row_a, [zv, ivc]
                )
                return carry

            lax.fori_loop(0, _CHUNK // _NL, body, 0, unroll=8)

        def pass_b(chunk):
            buf = outc[chunk % 2]

            def body(g, carry):
                base = chunk * _CHUNK + g * _NL
                iv = idx_v[pl.ds(base, _NL)]
                hi = iv >= _HALF
                iv2 = jnp.maximum(iv, _HALF) - _HALF
                vals = plsc.load_gather(row_b, [zv, iv2])
                cur = buf[0, pl.ds(g * _NL, _NL)]
                buf[0, pl.ds(g * _NL, _NL)] = jnp.where(hi, vals, cur)
                return carry

            lax.fori_loop(0, _CHUNK // _NL, body, 0, unroll=8)

        def wb_start(r, chunk):
            pltpu.make_async_copy(
                outc[chunk % 2],
                out_hbm.at[pl.ds(r, 1), pl.ds(chunk * _CHUNK, _CHUNK)],
                sem_w[chunk % 2],
            ).start()

        def wb_wait(r, chunk):
            pltpu.make_async_copy(
                outc[chunk % 2],
                out_hbm.at[pl.ds(r, 1), pl.ds(chunk * _CHUNK, _CHUNK)],
                sem_w[chunk % 2],
            ).wait()

        r0 = wid * rows_per_w
        r1 = r0 + 1

        pending = [None, None]

        def drain(buf_i):
            if pending[buf_i] is not None:
                wb_wait(*pending[buf_i])
                pending[buf_i] = None

        fire_a(r0)
        pltpu.sync_copy(idx_hbm, idx_v)
        for row_i, r in enumerate((r0, r1)):
            wait_a()
            fire_b(r)
            drain(0)
            pass_a(0)
            drain(1)
            pass_a(1)
            wait_b()
            pass_b(0)
            wb_start(r, 0)
            pending[0] = (r, 0)
            pass_b(1)
            wb_start(r, 1)
            pending[1] = (r, 1)
            drain(0)
            pass_a(2)
            drain(1)
            pass_a(3)
            if row_i == 0:
                fire_a(r1)
            pass_b(2)
            wb_start(r, 2)
            pending[0] = (r, 2)
            pass_b(3)
            wb_start(r, 3)
            pending[1] = (r, 3)
        drain(0)
        drain(1)

    return k(die_idx, table_t, tail_t)


def kernel(die_idx, die_embedding):
    table_t = die_embedding.T
    tail_t = jnp.pad(table_t[:, _TAIL0:], ((0, 0), (0, 128 - (_ROWS - _TAIL0))))
    out_t = _lookup(die_idx.astype(jnp.int32), table_t, tail_t)
    return out_t.T


# speedup vs baseline: 1.4702x; 1.4702x over previous
import functools

import jax
import jax.numpy as jnp
from jax import lax
from jax.experimental import pallas as pl
from jax.experimental.pallas import tpu as pltpu, tpu_sc as plsc

_BATCH = 16384
_DIM = 64
_ROWS = 100001
_SEG0 = 50048
_SEG1 = 49920
_TAIL0 = _SEG0 + _SEG1
_ROWBUF = _TAIL0 + 128
_CHUNK = 4096
_NCHUNK = _BATCH // _CHUNK
_NL = 16


@jax.jit
def _lookup(die_idx, table_t, tail_t):
    info = plsc.get_sparse_core_info()
    nw = info.num_cores * info.num_subcores
    rows_per_w = _DIM // nw
    assert rows_per_w == 2 and info.num_lanes == _NL

    mesh = plsc.VectorSubcoreMesh(core_axis_name="c", subcore_axis_name="s")

    @functools.partial(
        pl.kernel,
        mesh=mesh,
        compiler_params=pltpu.CompilerParams(needs_layout_passes=False),
        out_type=jax.ShapeDtypeStruct((_DIM, _BATCH), jnp.float32),
        scratch_types=[
            pltpu.VMEM((1, _ROWBUF), jnp.float32),
            pltpu.VMEM((_BATCH,), jnp.int32),
            pltpu.VMEM((1, _CHUNK), jnp.float32),
            pltpu.VMEM((1, _CHUNK), jnp.float32),
            pltpu.SemaphoreType.DMA,
            pltpu.SemaphoreType.DMA,
            pltpu.SemaphoreType.DMA,
        ],
    )
    def k(idx_hbm, table_hbm, tail_hbm, out_hbm, row_v, idx_v, out0, out1,
          sem_r, sem_w0, sem_w1):
        wid = lax.axis_index("s") * info.num_cores + lax.axis_index("c")
        zv = jnp.zeros((_NL,), jnp.int32)
        outc = (out0, out1)
        sem_w = (sem_w0, sem_w1)

        def row_descs(r):
            return (
                pltpu.make_async_copy(
                    table_hbm.at[pl.ds(r, 1), pl.ds(0, _SEG0)],
                    row_v.at[:, pl.ds(0, _SEG0)],
                    sem_r,
                ),
                pltpu.make_async_copy(
                    table_hbm.at[pl.ds(r, 1), pl.ds(_SEG0, _SEG1)],
                    row_v.at[:, pl.ds(_SEG0, _SEG1)],
                    sem_r,
                ),
                pltpu.make_async_copy(
                    tail_hbm.at[pl.ds(r, 1), :],
                    row_v.at[:, pl.ds(_TAIL0, 128)],
                    sem_r,
                ),
            )

        def fire_row(r):
            for d in row_descs(r):
                d.start()

        def wait_row():
            for d in row_descs(0):
                d.wait()

        def gather_chunk(chunk):
            buf = outc[chunk % 2]

            def body(g, carry):
                base = chunk * _CHUNK + g * _NL
                iv = idx_v[pl.ds(base, _NL)]
                buf[0, pl.ds(g * _NL, _NL)] = plsc.load_gather(
                    row_v, [zv, iv]
                )
                return carry

            lax.fori_loop(0, _CHUNK // _NL, body, 0, unroll=16)

        def wb_start(r, chunk):
            pltpu.make_async_copy(
                outc[chunk % 2],
                out_hbm.at[pl.ds(r, 1), pl.ds(chunk * _CHUNK, _CHUNK)],
                sem_w[chunk % 2],
            ).start()

        def wb_wait(r, chunk):
            pltpu.make_async_copy(
                outc[chunk % 2],
                out_hbm.at[pl.ds(r, 1), pl.ds(chunk * _CHUNK, _CHUNK)],
                sem_w[chunk % 2],
            ).wait()

        r0 = wid * rows_per_w
        pending = [None, None]

        def drain(buf_i):
            if pending[buf_i] is not None:
                wb_wait(*pending[buf_i])
                pending[buf_i] = None

        fire_row(r0)
        pltpu.sync_copy(idx_hbm, idx_v)
        for row_i in range(rows_per_w):
            r = r0 + row_i
            wait_row()
            for chunk in range(_NCHUNK):
                drain(chunk % 2)
                gather_chunk(chunk)
                wb_start(r, chunk)
                pending[chunk % 2] = (r, chunk)
            if row_i + 1 < rows_per_w:
                fire_row(r + 1)
        drain(0)
        drain(1)

    return k(die_idx, table_t, tail_t)


def kernel(die_idx, die_embedding):
    table_t = die_embedding.T
    tail_t = jnp.pad(table_t[:, _TAIL0:], ((0, 0), (0, 128 - (_ROWS - _TAIL0))))
    out_t = _lookup(die_idx.astype(jnp.int32), table_t, tail_t)
    return out_t.T


# speedup vs baseline: 1.9923x; 1.3551x over previous
import functools

import jax
import jax.numpy as jnp
from jax import lax
from jax.experimental import pallas as pl
from jax.experimental.pallas import tpu as pltpu, tpu_sc as plsc

_BATCH = 16384
_DIM = 64
_ROWS = 100001
_SEG0 = 50048
_SEG1 = 49920
_TAIL0 = _SEG0 + _SEG1
_ROWBUF = _TAIL0 + 128
_CHUNK = 4096
_NCHUNK = _BATCH // _CHUNK
_NL = 16


@jax.jit
def _lookup(die_idx, table_t, tail_t):
    info = plsc.get_sparse_core_info()
    nw = info.num_cores * info.num_subcores
    rows_per_w = _DIM // nw
    assert rows_per_w == 2 and info.num_lanes == _NL

    mesh = plsc.VectorSubcoreMesh(core_axis_name="c", subcore_axis_name="s")

    @functools.partial(
        pl.kernel,
        mesh=mesh,
        compiler_params=pltpu.CompilerParams(needs_layout_passes=False),
        out_type=jax.ShapeDtypeStruct((_DIM, _BATCH), jnp.float32),
        scratch_types=[
            pltpu.VMEM((1, _ROWBUF), jnp.float32),
            pltpu.VMEM((_BATCH,), jnp.int32),
            pltpu.VMEM((1, _CHUNK), jnp.float32),
            pltpu.VMEM((1, _CHUNK), jnp.float32),
            pltpu.SemaphoreType.DMA,
            pltpu.SemaphoreType.DMA,
            pltpu.SemaphoreType.DMA,
        ],
    )
    def k(idx_hbm, table_hbm, tail_hbm, out_hbm, row_v, idx_v, out0, out1,
          sem_r, sem_w0, sem_w1):
        wid = lax.axis_index("s") * info.num_cores + lax.axis_index("c")
        zv = jnp.zeros((_NL,), jnp.int32)
        outc = (out0, out1)
        sem_w = (sem_w0, sem_w1)

        def row_descs(r):
            return (
                pltpu.make_async_copy(
                    table_hbm.at[pl.ds(r, 1), pl.ds(0, _SEG0)],
                    row_v.at[:, pl.ds(0, _SEG0)],
                    sem_r,
                ),
                pltpu.make_async_copy(
                    table_hbm.at[pl.ds(r, 1), pl.ds(_SEG0, _SEG1)],
                    row_v.at[:, pl.ds(_SEG0, _SEG1)],
                    sem_r,
                ),
                pltpu.make_async_copy(
                    tail_hbm.at[pl.ds(r, 1), :],
                    row_v.at[:, pl.ds(_TAIL0, 128)],
                    sem_r,
                ),
            )

        def fire_row(r):
            for d in row_descs(r):
                d.start()

        def wait_row():
            for d in row_descs(0):
                d.wait()

        def gather_chunk(chunk):
            buf = outc[chunk % 2]
            gb_w = 8 * _NL

            def body(gb, carry):
                base = chunk * _CHUNK + gb * gb_w
                ivs = [
                    idx_v[pl.ds(base + j * _NL, _NL)] for j in range(8)
                ]
                vals = [plsc.load_gather(row_v, [zv, iv]) for iv in ivs]
                for j in range(8):
                    buf[0, pl.ds(gb * gb_w + j * _NL, _NL)] = vals[j]
                return carry

            lax.fori_loop(0, _CHUNK // gb_w, body, 0, unroll=2)

        def wb_start(r, chunk):
            pltpu.make_async_copy(
                outc[chunk % 2],
                out_hbm.at[pl.ds(r, 1), pl.ds(chunk * _CHUNK, _CHUNK)],
                sem_w[chunk % 2],
            ).start()

        def wb_wait(r, chunk):
            pltpu.make_async_copy(
                outc[chunk % 2],
                out_hbm.at[pl.ds(r, 1), pl.ds(chunk * _CHUNK, _CHUNK)],
                sem_w[chunk % 2],
            ).wait()

        r0 = wid * rows_per_w
        pending = [None, None]

        def drain(buf_i):
            if pending[buf_i] is not None:
                wb_wait(*pending[buf_i])
                pending[buf_i] = None

        fire_row(r0)
        pltpu.sync_copy(idx_hbm, idx_v)
        for row_i in range(rows_per_w):
            r = r0 + row_i
            wait_row()
            for chunk in range(_NCHUNK):
                drain(chunk % 2)
                gather_chunk(chunk)
                wb_start(r, chunk)
                pending[chunk % 2] = (r, chunk)
            if row_i + 1 < rows_per_w:
                fire_row(r + 1)
        drain(0)
        drain(1)

    return k(die_idx, table_t, tail_t)


def kernel(die_idx, die_embedding):
    table_t = die_embedding.T
    tail_t = jnp.pad(table_t[:, _TAIL0:], ((0, 0), (0, 128 - (_ROWS - _TAIL0))))
    out_t = _lookup(die_idx.astype(jnp.int32), table_t, tail_t)
    return out_t.T
